# baseline (speedup 1.0000x reference)
import functools

import jax
import jax.numpy as jnp
from jax import lax
from jax.experimental import pallas as pl
from jax.experimental.pallas import tpu as pltpu
from jax.experimental.pallas import tpu_sc as plsc

B = 16384
E = 64
NC = 2
NS = 16
NW = NC * NS
BPW = B // NW
NCH = BPW // 128

_mesh = plsc.VectorSubcoreMesh(
    core_axis_name="c", subcore_axis_name="s", num_cores=NC, num_subcores=NS
)


@functools.partial(
    pl.kernel,
    out_type=[
        jax.ShapeDtypeStruct((NW, 16), jnp.float32),
        jax.ShapeDtypeStruct((B // 128, 128), jnp.float32),
    ],
    mesh=_mesh,
    compiler_params=pltpu.CompilerParams(use_tc_tiling_on_sc=False),
    scratch_types=[
        pltpu.VMEM((NCH, 128), jnp.int32),
        pltpu.VMEM((NCH, 128), jnp.int32),
        pltpu.VMEM((BPW, E), jnp.float32),
        pltpu.VMEM((BPW, E), jnp.float32),
        pltpu.VMEM((NCH, 128), jnp.float32),
        pltpu.VMEM((NCH, 128), jnp.float32),
        pltpu.VMEM((NCH, 128), jnp.float32),
        pltpu.VMEM((16,), jnp.float32),
        pltpu.SemaphoreType.DMA,
        pltpu.SemaphoreType.DMA,
    ],
)
def _sc_gather_dot(
    uidx_hbm, midx_hbm, uemb_hbm, memb_hbm, ubias_hbm, mbias_hbm,
    part_hbm, bsum_hbm,
    uidx_v, midx_v, urows_v, mrows_v, ub_v, mb_v, bsum_v, part_v,
    sem_rows, sem_bias,
):
    wid = lax.axis_index("s") * NC + lax.axis_index("c")
    pltpu.sync_copy(uidx_hbm.at[pl.ds(wid * NCH, NCH)], uidx_v)
    pltpu.sync_copy(midx_hbm.at[pl.ds(wid * NCH, NCH)], midx_v)
    copies = []
    for j in range(NCH):
        copies.append(pltpu.async_copy(
            uemb_hbm.at[uidx_v.at[j]], urows_v.at[pl.ds(j * 128, 128)],
            sem_rows))
        copies.append(pltpu.async_copy(
            memb_hbm.at[midx_v.at[j]], mrows_v.at[pl.ds(j * 128, 128)],
            sem_rows))
        copies.append(pltpu.async_copy(
            ubias_hbm.at[uidx_v.at[j]], ub_v.at[j], sem_bias))
        copies.append(pltpu.async_copy(
            mbias_hbm.at[midx_v.at[j]], mb_v.at[j], sem_bias))
    for c in copies:
        c.wait()

    for j in range(NCH):
        for c in range(8):
            sl = pl.ds(c * 16, 16)
            bsum_v[j, sl] = ub_v[j, sl] + mb_v[j, sl]
    pltpu.sync_copy(bsum_v, bsum_hbm.at[pl.ds(wid * NCH, NCH)])

    def body(r, acc):
        for c in range(E // 16):
            sl = pl.ds(c * 16, 16)
            acc = acc + urows_v[r, sl] * mrows_v[r, sl]
        return acc

    acc = lax.fori_loop(0, BPW, body, jnp.zeros((16,), jnp.float32))
    part_v[...] = acc
    pltpu.sync_copy(part_v, part_hbm.at[wid])


def _tc_finish(part_ref, x_ref, o_ref):
    s = jnp.sum(part_ref[...])
    v = x_ref[...] + s
    o_ref[...] = 1.0 / (1.0 + jnp.exp(-v))


def kernel(inputs, user_embedding, user_bias, movie_embedding, movie_bias):
    uidx = inputs[:, 0].reshape(B // 128, 128)
    midx = inputs[:, 1].reshape(B // 128, 128)
    ub_flat = user_bias.reshape(-1)
    mb_flat = movie_bias.reshape(-1)
    partials, bsum = _sc_gather_dot(
        uidx, midx, user_embedding, movie_embedding, ub_flat, mb_flat)
    out = pl.pallas_call(
        _tc_finish,
        out_shape=jax.ShapeDtypeStruct((B // 128, 128), jnp.float32),
    )(partials, bsum)
    return out.reshape(B, 1)


# speedup vs baseline: 1.2280x; 1.2280x over previous
import functools

import jax
import jax.numpy as jnp
from jax import lax
from jax.experimental import pallas as pl
from jax.experimental.pallas import tpu as pltpu
from jax.experimental.pallas import tpu_sc as plsc

B = 16384
E = 64
NC = 2
NS = 16
NW = NC * NS
BPW = B // NW
CH = 128
NCHUNK = BPW // CH

_mesh = plsc.VectorSubcoreMesh(
    core_axis_name="c", subcore_axis_name="s", num_cores=NC, num_subcores=NS
)


@functools.partial(
    pl.kernel,
    out_type=[
        jax.ShapeDtypeStruct((NW, 16), jnp.float32),
        jax.ShapeDtypeStruct((B // 128, 128), jnp.float32),
    ],
    mesh=_mesh,
    compiler_params=pltpu.CompilerParams(needs_layout_passes=False),
    scratch_types=[
        pltpu.VMEM((NCHUNK, 128), jnp.int32),
        pltpu.VMEM((NCHUNK, 128), jnp.int32),
        pltpu.VMEM((2 * CH, E), jnp.float32),
        pltpu.VMEM((2 * CH, E), jnp.float32),
        pltpu.VMEM((BPW * 8,), jnp.float32),
        pltpu.VMEM((BPW * 8,), jnp.float32),
        pltpu.VMEM((NCHUNK, 128), jnp.float32),
        pltpu.VMEM((16,), jnp.float32),
        pltpu.SemaphoreType.DMA,
        pltpu.SemaphoreType.DMA,
        pltpu.SemaphoreType.DMA,
    ],
)
def _sc_gather_dot(
    uidx_hbm, midx_hbm, uemb_hbm, memb_hbm, ubias_hbm, mbias_hbm,
    part_hbm, bsum_hbm,
    uidx_v, midx_v, urows_v, mrows_v, ub_v, mb_v, bsum_v, part_v,
    sem_a, sem_b, sem_bias,
):
    wid = lax.axis_index("s") * NC + lax.axis_index("c")
    pltpu.sync_copy(uidx_hbm.at[pl.ds(wid * NCHUNK, NCHUNK)], uidx_v)
    pltpu.sync_copy(midx_hbm.at[pl.ds(wid * NCHUNK, NCHUNK)], midx_v)

    sems = [sem_a, sem_b]

    def fire_chunk(k):
        boff = (k % 2) * CH
        sem = sems[k % 2]

        def f(g, _):
            ju = uidx_v[k, pl.ds(g * 16, 16)]
            jm = midx_v[k, pl.ds(g * 16, 16)]
            for l in range(16):
                iu = ju[l]
                im = jm[l]
                d = boff + g * 16 + l
                r = k * CH + g * 16 + l
                pltpu.async_copy(
                    uemb_hbm.at[pl.ds(iu, 1)], urows_v.at[pl.ds(d, 1)], sem)
                pltpu.async_copy(
                    memb_hbm.at[pl.ds(im, 1)], mrows_v.at[pl.ds(d, 1)], sem)
                pltpu.async_copy(
                    ubias_hbm.at[pl.ds((iu // 8) * 8, 8)],
                    ub_v.at[pl.ds(r * 8, 8)], sem_bias)
                pltpu.async_copy(
                    mbias_hbm.at[pl.ds((im // 8) * 8, 8)],
                    mb_v.at[pl.ds(r * 8, 8)], sem_bias)
            return 0

        lax.fori_loop(0, CH // 16, f, 0)

    def drain_rows(k):
        sem = sems[k % 2]

        def d(_, carry):
            pltpu.make_async_copy(
                uemb_hbm.at[pl.ds(0, 1)], urows_v.at[pl.ds(0, 1)], sem).wait()
            pltpu.make_async_copy(
                uemb_hbm.at[pl.ds(0, 1)], urows_v.at[pl.ds(0, 1)], sem).wait()
            return carry

        lax.fori_loop(0, CH, d, 0)

    def compute_chunk(k, acc):
        boff = (k % 2) * CH

        def body(rr, a):
            for c in range(E // 16):
                sl = pl.ds(c * 16, 16)
                a = a + urows_v[boff + rr, sl] * mrows_v[boff + rr, sl]
            return a

        return lax.fori_loop(0, CH, body, acc)

    fire_chunk(0)
    acc = jnp.zeros((16,), jnp.float32)
    for k in range(NCHUNK):
        if k + 1 < NCHUNK:
            fire_chunk(k + 1)
        drain_rows(k)
        acc = compute_chunk(k, acc)

    part_v[...] = acc
    pltpu.sync_copy(part_v, part_hbm.at[wid])

    def drain_bias(_, carry):
        pltpu.make_async_copy(
            ubias_hbm.at[pl.ds(0, 8)], ub_v.at[pl.ds(0, 8)], sem_bias).wait()
        pltpu.make_async_copy(
            ubias_hbm.at[pl.ds(0, 8)], ub_v.at[pl.ds(0, 8)], sem_bias).wait()
        return carry

    lax.fori_loop(0, BPW, drain_bias, 0)

    lanes8 = lax.iota(jnp.int32, 16) * 8
    for j in range(NCHUNK):
        for c in range(8):
            sl = pl.ds(c * 16, 16)
            ju = uidx_v[j, sl]
            jm = midx_v[j, sl]
            base = (j * 128 + c * 16) * 8 + lanes8
            uvals = plsc.load_gather(ub_v, [base + (ju % 8)])
            mvals = plsc.load_gather(mb_v, [base + (jm % 8)])
            bsum_v[j, sl] = uvals + mvals
    pltpu.sync_copy(bsum_v, bsum_hbm.at[pl.ds(wid * NCHUNK, NCHUNK)])


def _tc_finish(part_ref, x_ref, o_ref):
    s = jnp.sum(part_ref[...])
    v = x_ref[...] + s
    o_ref[...] = 1.0 / (1.0 + jnp.exp(-v))


def kernel(inputs, user_embedding, user_bias, movie_embedding, movie_bias):
    uidx = inputs[:, 0].reshape(B // 128, 128)
    midx = inputs[:, 1].reshape(B // 128, 128)
    partials, bsum = _sc_gather_dot(
        uidx, midx, user_embedding, movie_embedding,
        user_bias.reshape(-1), movie_bias.reshape(-1))
    out = pl.pallas_call(
        _tc_finish,
        out_shape=jax.ShapeDtypeStruct((B // 128, 128), jnp.float32),
    )(partials, bsum)
    return out.reshape(B, 1)
